# Initial kernel scaffold; baseline (speedup 1.0000x reference)
import jax
import jax.numpy as jnp
from jax.experimental import pallas as pl


def kernel(positions, species, edge_index, W_embed, E0, Wr1, Wr2, Wr3, Wup, Wout, Wskip, Wprod, WprodLin, Wro, Wh, Wo):
    raise NotImplementedError("write your pallas kernel here")


# speedup vs baseline: 2.1993x; 2.1993x over previous
import functools

import jax
import jax.numpy as jnp
import numpy as np
from jax import lax
from jax.experimental import pallas as pl
from jax.experimental.pallas import tpu as pltpu
from jax.experimental.pallas import tpu_sc as plsc

N = 10000
E = 320000
D = 128
Z = 10
NB = 8
RMAX = 5.0
P = 5
H1 = 64
MLPH = 16
NI = 2
AVG = 32.0

NC = 2
NS = 16
NW = NC * NS
EPW = E // NW
EB = 80
NPT = N // NS

_mesh = plsc.VectorSubcoreMesh(core_axis_name="c", subcore_axis_name="s")


@functools.partial(
    pl.kernel,
    out_type=jax.ShapeDtypeStruct((E,), jnp.float32),
    mesh=_mesh,
    scratch_types=[
        pltpu.VMEM((N,), jnp.float32),
        pltpu.VMEM((N,), jnp.float32),
        pltpu.VMEM((N,), jnp.float32),
        pltpu.VMEM((EB,), jnp.int32),
        pltpu.VMEM((EB,), jnp.int32),
        pltpu.VMEM((EB,), jnp.float32),
    ],
    compiler_params=pltpu.CompilerParams(needs_layout_passes=False, use_tc_tiling_on_sc=False),
)
def _geom(px_hbm, py_hbm, pz_hbm, src_hbm, dst_hbm, r2_hbm,
          px_v, py_v, pz_v, si_v, di_v, out_v):
    wid = lax.axis_index("s") * NC + lax.axis_index("c")
    pltpu.sync_copy(px_hbm, px_v)
    pltpu.sync_copy(py_hbm, py_v)
    pltpu.sync_copy(pz_hbm, pz_v)
    base0 = wid * EPW

    def blk(b, carry):
        base = base0 + b * EB
        pltpu.sync_copy(src_hbm.at[pl.ds(base, EB)], si_v)
        pltpu.sync_copy(dst_hbm.at[pl.ds(base, EB)], di_v)

        def sub(j, c2):
            s16 = si_v[pl.ds(j * 16, 16)]
            d16 = di_v[pl.ds(j * 16, 16)]
            dx = plsc.load_gather(px_v, [s16]) - plsc.load_gather(px_v, [d16])
            dy = plsc.load_gather(py_v, [s16]) - plsc.load_gather(py_v, [d16])
            dz = plsc.load_gather(pz_v, [s16]) - plsc.load_gather(pz_v, [d16])
            out_v[pl.ds(j * 16, 16)] = dx * dx + dy * dy + dz * dz
            return c2

        lax.fori_loop(0, EB // 16, sub, 0)
        pltpu.sync_copy(out_v, r2_hbm.at[pl.ds(base, EB)])
        return carry

    lax.fori_loop(0, EPW // EB, blk, 0)


@functools.partial(
    pl.kernel,
    out_type=jax.ShapeDtypeStruct((NC, N, D), jnp.float32),
    mesh=_mesh,
    scratch_types=[
        pltpu.VMEM((EB,), jnp.int32),
        pltpu.VMEM((EB,), jnp.int32),
        pltpu.VMEM((EB, D), jnp.float32),
        pltpu.VMEM((EB, D), jnp.float32),
        pltpu.VMEM_SHARED((N, D), jnp.float32),
        pltpu.SemaphoreType.DMA,
    ],
    compiler_params=pltpu.CompilerParams(needs_layout_passes=False, use_tc_tiling_on_sc=False),
)
def _gather_mul_scatter(ew_hbm, hs_hbm, src_hbm, dst_hbm, zeros_hbm, out_hbm,
                        si_v, di_v, hsg_v, ew_v, agg_sh, sem):
    c = lax.axis_index("c")
    s = lax.axis_index("s")
    wid = s * NC + c
    pltpu.sync_copy(zeros_hbm.at[pl.ds(s * NPT, NPT)],
                    agg_sh.at[pl.ds(s * NPT, NPT)])
    plsc.subcore_barrier()
    base0 = wid * EPW

    def blk(b, carry):
        base = base0 + b * EB
        pltpu.sync_copy(src_hbm.at[pl.ds(base, EB)], si_v)
        pltpu.sync_copy(dst_hbm.at[pl.ds(base, EB)], di_v)
        gcp = pltpu.async_copy(hs_hbm.at[si_v], hsg_v, sem)
        pltpu.sync_copy(ew_hbm.at[pl.ds(base, EB)], ew_v)
        gcp.wait()

        def row(rr, c2):
            for ch in range(D // 16):
                sl = pl.ds(ch * 16, 16)
                ew_v[rr, sl] = ew_v[rr, sl] * hsg_v[rr, sl]
            return c2

        lax.fori_loop(0, EB, row, 0)
        pltpu.sync_copy(ew_v, agg_sh.at[di_v], add=True)
        return carry

    lax.fori_loop(0, EPW // EB, blk, 0)
    plsc.subcore_barrier()
    pltpu.sync_copy(agg_sh.at[pl.ds(s * NPT, NPT)],
                    out_hbm.at[c, pl.ds(s * NPT, NPT)])


def _edge_mlp_body(r2_ref, wr1_ref, wr2_ref, wr3_ref, ew0_ref, ew1_ref):
    r2 = r2_ref[:, :]
    r = jnp.sqrt(r2)
    rs = jnp.maximum(r, 1e-9)
    n = (lax.broadcasted_iota(jnp.int32, (1, NB), 1) + 1).astype(jnp.float32)
    rb = (np.float32(np.sqrt(2.0 / RMAX))
          * jnp.sin(rs * (np.pi / RMAX) * n) / rs)
    x = r * np.float32(1.0 / RMAX)
    x2 = x * x
    x4 = x2 * x2
    x5 = x4 * x
    env = (1.0 - 21.0 * x5 + 35.0 * x5 * x - 15.0 * x5 * x2)
    env = jnp.where(x < 1.0, env, 0.0)
    rb = rb * env
    outs = (ew0_ref, ew1_ref)
    for i in range(NI):
        t = rb @ wr1_ref[i]
        t = t * jax.nn.sigmoid(t)
        t = t @ wr2_ref[i]
        t = t * jax.nn.sigmoid(t)
        outs[i][:, :] = t @ wr3_ref[i]


def _edge_mlp(r2, Wr1, Wr2, Wr3, be=512):
    grid = E // be
    return pl.pallas_call(
        _edge_mlp_body,
        grid=(grid,),
        in_specs=[
            pl.BlockSpec((be, 1), lambda i: (i, 0)),
            pl.BlockSpec((NI, NB, H1), lambda i: (0, 0, 0)),
            pl.BlockSpec((NI, H1, H1), lambda i: (0, 0, 0)),
            pl.BlockSpec((NI, H1, D), lambda i: (0, 0, 0)),
        ],
        out_specs=[
            pl.BlockSpec((be, D), lambda i: (i, 0)),
            pl.BlockSpec((be, D), lambda i: (i, 0)),
        ],
        out_shape=[
            jax.ShapeDtypeStruct((E, D), jnp.float32),
            jax.ShapeDtypeStruct((E, D), jnp.float32),
        ],
    )(r2, Wr1, Wr2, Wr3)


def _node_init_body(sp_ref, wemb_ref, e0_ref, wup0_ref, h_ref, hs_ref, en_ref):
    sp = sp_ref[:, :]
    zi = lax.broadcasted_iota(jnp.int32, (1, Z), 1)
    oh = (sp == zi).astype(jnp.float32)
    h = oh @ wemb_ref[:, :]
    h_ref[:, :] = h
    hs_ref[:, :] = h @ wup0_ref[:, :]
    en_ref[:, :] = oh @ e0_ref[:, :]


def _node_init(species2, W_embed, E0c, Wup0, bn=2000):
    grid = N // bn
    return pl.pallas_call(
        _node_init_body,
        grid=(grid,),
        in_specs=[
            pl.BlockSpec((bn, 1), lambda i: (i, 0)),
            pl.BlockSpec((Z, D), lambda i: (0, 0)),
            pl.BlockSpec((Z, 1), lambda i: (0, 0)),
            pl.BlockSpec((D, D), lambda i: (0, 0)),
        ],
        out_specs=[
            pl.BlockSpec((bn, D), lambda i: (i, 0)),
            pl.BlockSpec((bn, D), lambda i: (i, 0)),
            pl.BlockSpec((bn, 1), lambda i: (i, 0)),
        ],
        out_shape=[
            jax.ShapeDtypeStruct((N, D), jnp.float32),
            jax.ShapeDtypeStruct((N, D), jnp.float32),
            jax.ShapeDtypeStruct((N, 1), jnp.float32),
        ],
    )(species2, W_embed, E0c, Wup0)


def _node_update_body(final, agg_ref, h_ref, sp_ref, en_ref, wout_ref,
                      wskip_ref, wprod_ref, wpl_ref, wro_ref, wup_ref,
                      wh_ref, wo_ref, h_out, hs_out, en_out):
    agg = (agg_ref[0] + agg_ref[1]) * np.float32(1.0 / AVG)
    h1 = agg @ wout_ref[:, :] + h_ref[:, :] @ wskip_ref[:, :]
    sp = sp_ref[:, :]
    zi = lax.broadcasted_iota(jnp.int32, (1, Z), 1)
    oh = (sp == zi).astype(jnp.float32)
    w = oh @ wprod_ref[:, :]
    g = (w[:, 0:D] * h1 + w[:, D:2 * D] * (h1 * h1)
         + w[:, 2 * D:3 * D] * (h1 * h1 * h1))
    h2 = g @ wpl_ref[:, :]
    h_out[:, :] = h2
    if final:
        t = h2 @ wh_ref[:, :]
        t = t * jax.nn.sigmoid(t)
        en_out[:, :] = en_ref[:, :] + t @ wo_ref[:, :]
        hs_out[:, :] = h2
    else:
        en_out[:, :] = en_ref[:, :] + h2 @ wro_ref[:, :]
        hs_out[:, :] = h2 @ wup_ref[:, :]


def _node_update(final, agg2, h, species2, en, Wout_i, Wskip_i, WprodF_i,
                 WprodLin_i, Wro_i, Wup_n, Wh, Wo, bn=2000):
    grid = N // bn
    return pl.pallas_call(
        functools.partial(_node_update_body, final),
        grid=(grid,),
        in_specs=[
            pl.BlockSpec((NC, bn, D), lambda i: (0, i, 0)),
            pl.BlockSpec((bn, D), lambda i: (i, 0)),
            pl.BlockSpec((bn, 1), lambda i: (i, 0)),
            pl.BlockSpec((bn, 1), lambda i: (i, 0)),
            pl.BlockSpec((D, D), lambda i: (0, 0)),
            pl.BlockSpec((D, D), lambda i: (0, 0)),
            pl.BlockSpec((Z, 3 * D), lambda i: (0, 0)),
            pl.BlockSpec((D, D), lambda i: (0, 0)),
            pl.BlockSpec((D, 1), lambda i: (0, 0)),
            pl.BlockSpec((D, D), lambda i: (0, 0)),
            pl.BlockSpec((D, MLPH), lambda i: (0, 0)),
            pl.BlockSpec((MLPH, 1), lambda i: (0, 0)),
        ],
        out_specs=[
            pl.BlockSpec((bn, D), lambda i: (i, 0)),
            pl.BlockSpec((bn, D), lambda i: (i, 0)),
            pl.BlockSpec((bn, 1), lambda i: (i, 0)),
        ],
        out_shape=[
            jax.ShapeDtypeStruct((N, D), jnp.float32),
            jax.ShapeDtypeStruct((N, D), jnp.float32),
            jax.ShapeDtypeStruct((N, 1), jnp.float32),
        ],
    )(agg2, h, species2, en, Wout_i, Wskip_i, WprodF_i, WprodLin_i,
      Wro_i, Wup_n, Wh, Wo)


def kernel(positions, species, edge_index, W_embed, E0, Wr1, Wr2, Wr3,
           Wup, Wout, Wskip, Wprod, WprodLin, Wro, Wh, Wo):
    src = edge_index[0]
    dst = edge_index[1]
    px = positions[:, 0]
    py = positions[:, 1]
    pz = positions[:, 2]
    species2 = species.reshape(N, 1).astype(jnp.int32)
    zeros_nd = jnp.zeros((N, D), jnp.float32)

    r2 = _geom(px, py, pz, src, dst)
    ew0, ew1 = _edge_mlp(r2.reshape(E, 1), Wr1, Wr2, Wr3)
    h, hs, en = _node_init(species2, W_embed, E0.reshape(Z, 1), Wup[0])

    ews = (ew0, ew1)
    for i in range(NI):
        agg2 = _gather_mul_scatter(ews[i], hs, src, dst, zeros_nd)
        h, hs, en = _node_update(
            i == NI - 1, agg2, h, species2, en,
            Wout[i], Wskip[i], Wprod[i].reshape(Z, 3 * D), WprodLin[i],
            Wro[i], Wup[(i + 1) % NI], Wh, Wo)
    return en.reshape(N)


# speedup vs baseline: 4.3492x; 1.9775x over previous
import functools

import jax
import jax.numpy as jnp
import numpy as np
from jax import lax
from jax.experimental import pallas as pl
from jax.experimental.pallas import tpu as pltpu
from jax.experimental.pallas import tpu_sc as plsc

N = 10000
E = 320000
D = 128
Z = 10
NB = 8
RMAX = 5.0
H1 = 64
MLPH = 16
NI = 2
AVG = 32.0

K = 16384
KTAB = 16896
TAB_H = np.float32(5.25 / K)
INV_H = np.float32(K / 5.25)

NC = 2
NS = 16
NW = NC * NS
EPW = E // NW
EBG = 2000
EB = 80
NPT = N // NS

_mesh = plsc.VectorSubcoreMesh(core_axis_name="c", subcore_axis_name="s")
_sc_params = pltpu.CompilerParams(needs_layout_passes=False,
                                  use_tc_tiling_on_sc=False)


@functools.partial(
    pl.kernel,
    out_type=jax.ShapeDtypeStruct((E,), jnp.int32),
    mesh=_mesh,
    scratch_types=[
        pltpu.VMEM((N,), jnp.float32),
        pltpu.VMEM((N,), jnp.float32),
        pltpu.VMEM((N,), jnp.float32),
        pltpu.VMEM((EBG,), jnp.int32),
        pltpu.VMEM((EBG,), jnp.int32),
        pltpu.VMEM((EBG,), jnp.int32),
    ],
    compiler_params=_sc_params,
)
def _geom(px_hbm, py_hbm, pz_hbm, src_hbm, dst_hbm, ki_hbm,
          px_v, py_v, pz_v, si_v, di_v, ko_v):
    wid = lax.axis_index("s") * NC + lax.axis_index("c")
    pltpu.sync_copy(px_hbm, px_v)
    pltpu.sync_copy(py_hbm, py_v)
    pltpu.sync_copy(pz_hbm, pz_v)
    base0 = wid * EPW

    def blk(b, carry):
        base = base0 + b * EBG
        pltpu.sync_copy(src_hbm.at[pl.ds(base, EBG)], si_v)
        pltpu.sync_copy(dst_hbm.at[pl.ds(base, EBG)], di_v)

        def sub(j, c2):
            s16 = si_v[pl.ds(j * 16, 16)]
            d16 = di_v[pl.ds(j * 16, 16)]
            dx = plsc.load_gather(px_v, [s16]) - plsc.load_gather(px_v, [d16])
            dy = plsc.load_gather(py_v, [s16]) - plsc.load_gather(py_v, [d16])
            dz = plsc.load_gather(pz_v, [s16]) - plsc.load_gather(pz_v, [d16])
            r2 = jnp.maximum(dx * dx + dy * dy + dz * dz, 1e-24)
            ii = plsc.bitcast(r2, jnp.int32)
            ii = jnp.int32(0x5F3759DF) - lax.shift_right_logical(ii, 1)
            y = plsc.bitcast(ii, jnp.float32)
            y = y * (1.5 - 0.5 * r2 * y * y)
            y = y * (1.5 - 0.5 * r2 * y * y)
            y = y * (1.5 - 0.5 * r2 * y * y)
            u = (r2 * y) * INV_H + 0.5
            k = jnp.minimum(u.astype(jnp.int32), KTAB - 1)
            ko_v[pl.ds(j * 16, 16)] = k
            return c2

        lax.fori_loop(0, EBG // 16, sub, 0)
        pltpu.sync_copy(ko_v, ki_hbm.at[pl.ds(base, EBG)])
        return carry

    lax.fori_loop(0, EPW // EBG, blk, 0)


@functools.partial(
    pl.kernel,
    out_type=jax.ShapeDtypeStruct((NC, N, D), jnp.float32),
    mesh=_mesh,
    scratch_types=[
        pltpu.VMEM((EB,), jnp.int32),
        pltpu.VMEM((EB,), jnp.int32),
        pltpu.VMEM((EB,), jnp.int32),
        pltpu.VMEM((EB, D), jnp.float32),
        pltpu.VMEM((EB, D), jnp.float32),
        pltpu.VMEM_SHARED((N, D), jnp.float32),
        pltpu.SemaphoreType.DMA,
        pltpu.SemaphoreType.DMA,
    ],
    compiler_params=_sc_params,
)
def _gather_mul_scatter(tab_hbm, hs_hbm, ki_hbm, src_hbm, dst_hbm, zeros_hbm,
                        out_hbm, ki_v, si_v, di_v, ta_v, hsg_v, agg_sh,
                        sem1, sem2):
    c = lax.axis_index("c")
    s = lax.axis_index("s")
    wid = s * NC + c
    pltpu.sync_copy(zeros_hbm.at[pl.ds(s * NPT, NPT)],
                    agg_sh.at[pl.ds(s * NPT, NPT)])
    plsc.subcore_barrier()
    base0 = wid * EPW

    def blk(b, carry):
        base = base0 + b * EB
        pltpu.sync_copy(ki_hbm.at[pl.ds(base, EB)], ki_v)
        pltpu.sync_copy(src_hbm.at[pl.ds(base, EB)], si_v)
        pltpu.sync_copy(dst_hbm.at[pl.ds(base, EB)], di_v)
        tcp = pltpu.async_copy(tab_hbm.at[ki_v], ta_v, sem1)
        hcp = pltpu.async_copy(hs_hbm.at[si_v], hsg_v, sem2)
        tcp.wait()
        hcp.wait()

        def row(rr, c2):
            for ch in range(D // 16):
                sl = pl.ds(ch * 16, 16)
                ta_v[rr, sl] = ta_v[rr, sl] * hsg_v[rr, sl]
            return c2

        lax.fori_loop(0, EB, row, 0)
        pltpu.sync_copy(ta_v, agg_sh.at[di_v], add=True)
        return carry

    lax.fori_loop(0, EPW // EB, blk, 0)
    plsc.subcore_barrier()
    pltpu.sync_copy(agg_sh.at[pl.ds(s * NPT, NPT)],
                    out_hbm.at[c, pl.ds(s * NPT, NPT)])


def _table_body(wr1_ref, wr2_ref, wr3_ref, t0_ref, t1_ref):
    i = pl.program_id(0)
    row0 = i * 512
    ridx = (lax.broadcasted_iota(jnp.int32, (512, 1), 0) + row0)
    r = ridx.astype(jnp.float32) * TAB_H
    rs = jnp.maximum(r, 1e-9)
    n = (lax.broadcasted_iota(jnp.int32, (1, NB), 1) + 1).astype(jnp.float32)
    rb = (np.float32(np.sqrt(2.0 / RMAX))
          * jnp.sin(rs * (np.pi / RMAX) * n) / rs)
    x = r * np.float32(1.0 / RMAX)
    x2 = x * x
    x5 = x2 * x2 * x
    env = (1.0 - 21.0 * x5 + 35.0 * x5 * x - 15.0 * x5 * x2)
    env = jnp.where(x < 1.0, env, 0.0)
    rb = rb * env
    outs = (t0_ref, t1_ref)
    for i2 in range(NI):
        t = rb @ wr1_ref[i2]
        t = t * jax.nn.sigmoid(t)
        t = t @ wr2_ref[i2]
        t = t * jax.nn.sigmoid(t)
        outs[i2][:, :] = t @ wr3_ref[i2]


def _table(Wr1, Wr2, Wr3):
    return pl.pallas_call(
        _table_body,
        grid=(KTAB // 512,),
        in_specs=[
            pl.BlockSpec((NI, NB, H1), lambda i: (0, 0, 0)),
            pl.BlockSpec((NI, H1, H1), lambda i: (0, 0, 0)),
            pl.BlockSpec((NI, H1, D), lambda i: (0, 0, 0)),
        ],
        out_specs=[
            pl.BlockSpec((512, D), lambda i: (i, 0)),
            pl.BlockSpec((512, D), lambda i: (i, 0)),
        ],
        out_shape=[
            jax.ShapeDtypeStruct((KTAB, D), jnp.float32),
            jax.ShapeDtypeStruct((KTAB, D), jnp.float32),
        ],
    )(Wr1, Wr2, Wr3)


def _node_init_body(sp_ref, wemb_ref, e0_ref, wup0_ref, h_ref, hs_ref, en_ref):
    sp = sp_ref[:, :]
    zi = lax.broadcasted_iota(jnp.int32, (1, Z), 1)
    oh = (sp == zi).astype(jnp.float32)
    h = oh @ wemb_ref[:, :]
    h_ref[:, :] = h
    hs_ref[:, :] = h @ wup0_ref[:, :]
    en_ref[:, :] = oh @ e0_ref[:, :]


def _node_init(species2, W_embed, E0c, Wup0, bn=2000):
    grid = N // bn
    return pl.pallas_call(
        _node_init_body,
        grid=(grid,),
        in_specs=[
            pl.BlockSpec((bn, 1), lambda i: (i, 0)),
            pl.BlockSpec((Z, D), lambda i: (0, 0)),
            pl.BlockSpec((Z, 1), lambda i: (0, 0)),
            pl.BlockSpec((D, D), lambda i: (0, 0)),
        ],
        out_specs=[
            pl.BlockSpec((bn, D), lambda i: (i, 0)),
            pl.BlockSpec((bn, D), lambda i: (i, 0)),
            pl.BlockSpec((bn, 1), lambda i: (i, 0)),
        ],
        out_shape=[
            jax.ShapeDtypeStruct((N, D), jnp.float32),
            jax.ShapeDtypeStruct((N, D), jnp.float32),
            jax.ShapeDtypeStruct((N, 1), jnp.float32),
        ],
    )(species2, W_embed, E0c, Wup0)


def _node_update_body(final, agg_ref, h_ref, sp_ref, en_ref, wout_ref,
                      wskip_ref, wprod_ref, wpl_ref, wro_ref, wup_ref,
                      wh_ref, wo_ref, h_out, hs_out, en_out):
    agg = (agg_ref[0] + agg_ref[1]) * np.float32(1.0 / AVG)
    h1 = agg @ wout_ref[:, :] + h_ref[:, :] @ wskip_ref[:, :]
    sp = sp_ref[:, :]
    zi = lax.broadcasted_iota(jnp.int32, (1, Z), 1)
    oh = (sp == zi).astype(jnp.float32)
    w = oh @ wprod_ref[:, :]
    g = (w[:, 0:D] * h1 + w[:, D:2 * D] * (h1 * h1)
         + w[:, 2 * D:3 * D] * (h1 * h1 * h1))
    h2 = g @ wpl_ref[:, :]
    h_out[:, :] = h2
    if final:
        t = h2 @ wh_ref[:, :]
        t = t * jax.nn.sigmoid(t)
        en_out[:, :] = en_ref[:, :] + t @ wo_ref[:, :]
        hs_out[:, :] = h2
    else:
        en_out[:, :] = en_ref[:, :] + h2 @ wro_ref[:, :]
        hs_out[:, :] = h2 @ wup_ref[:, :]


def _node_update(final, agg2, h, species2, en, Wout_i, Wskip_i, WprodF_i,
                 WprodLin_i, Wro_i, Wup_n, Wh, Wo, bn=2000):
    grid = N // bn
    return pl.pallas_call(
        functools.partial(_node_update_body, final),
        grid=(grid,),
        in_specs=[
            pl.BlockSpec((NC, bn, D), lambda i: (0, i, 0)),
            pl.BlockSpec((bn, D), lambda i: (i, 0)),
            pl.BlockSpec((bn, 1), lambda i: (i, 0)),
            pl.BlockSpec((bn, 1), lambda i: (i, 0)),
            pl.BlockSpec((D, D), lambda i: (0, 0)),
            pl.BlockSpec((D, D), lambda i: (0, 0)),
            pl.BlockSpec((Z, 3 * D), lambda i: (0, 0)),
            pl.BlockSpec((D, D), lambda i: (0, 0)),
            pl.BlockSpec((D, 1), lambda i: (0, 0)),
            pl.BlockSpec((D, D), lambda i: (0, 0)),
            pl.BlockSpec((D, MLPH), lambda i: (0, 0)),
            pl.BlockSpec((MLPH, 1), lambda i: (0, 0)),
        ],
        out_specs=[
            pl.BlockSpec((bn, D), lambda i: (i, 0)),
            pl.BlockSpec((bn, D), lambda i: (i, 0)),
            pl.BlockSpec((bn, 1), lambda i: (i, 0)),
        ],
        out_shape=[
            jax.ShapeDtypeStruct((N, D), jnp.float32),
            jax.ShapeDtypeStruct((N, D), jnp.float32),
            jax.ShapeDtypeStruct((N, 1), jnp.float32),
        ],
    )(agg2, h, species2, en, Wout_i, Wskip_i, WprodF_i, WprodLin_i,
      Wro_i, Wup_n, Wh, Wo)


def kernel(positions, species, edge_index, W_embed, E0, Wr1, Wr2, Wr3,
           Wup, Wout, Wskip, Wprod, WprodLin, Wro, Wh, Wo):
    src = edge_index[0]
    dst = edge_index[1]
    px = positions[:, 0]
    py = positions[:, 1]
    pz = positions[:, 2]
    species2 = species.reshape(N, 1).astype(jnp.int32)
    zeros_nd = jnp.zeros((N, D), jnp.float32)

    kidx = _geom(px, py, pz, src, dst)
    t0, t1 = _table(Wr1, Wr2, Wr3)
    h, hs, en = _node_init(species2, W_embed, E0.reshape(Z, 1), Wup[0])

    tabs = (t0, t1)
    for i in range(NI):
        agg2 = _gather_mul_scatter(tabs[i], hs, kidx, src, dst, zeros_nd)
        h, hs, en = _node_update(
            i == NI - 1, agg2, h, species2, en,
            Wout[i], Wskip[i], Wprod[i].reshape(Z, 3 * D), WprodLin[i],
            Wro[i], Wup[(i + 1) % NI], Wh, Wo)
    return en.reshape(N)


# speedup vs baseline: 4.6742x; 1.0747x over previous
import functools

import jax
import jax.numpy as jnp
import numpy as np
from jax import lax
from jax.experimental import pallas as pl
from jax.experimental.pallas import tpu as pltpu
from jax.experimental.pallas import tpu_sc as plsc

N = 10000
E = 320000
D = 128
Z = 10
NB = 8
RMAX = 5.0
H1 = 64
MLPH = 16
NI = 2
AVG = 32.0

K = 16384
KTAB = 16896
TAB_H = np.float32(5.25 / K)
INV_H = np.float32(K / 5.25)

NC = 2
NS = 16
NW = NC * NS
EPW = E // NW
EBG = 2000
EB = 80
NPT = N // NS

_mesh = plsc.VectorSubcoreMesh(core_axis_name="c", subcore_axis_name="s")
_sc_params = pltpu.CompilerParams(needs_layout_passes=False,
                                  use_tc_tiling_on_sc=False)


@functools.partial(
    pl.kernel,
    out_type=jax.ShapeDtypeStruct((E,), jnp.int32),
    mesh=_mesh,
    scratch_types=[
        pltpu.VMEM((N,), jnp.float32),
        pltpu.VMEM((N,), jnp.float32),
        pltpu.VMEM((N,), jnp.float32),
        pltpu.VMEM((EBG,), jnp.int32),
        pltpu.VMEM((EBG,), jnp.int32),
        pltpu.VMEM((EBG,), jnp.int32),
    ],
    compiler_params=_sc_params,
)
def _geom(px_hbm, py_hbm, pz_hbm, src_hbm, dst_hbm, ki_hbm,
          px_v, py_v, pz_v, si_v, di_v, ko_v):
    wid = lax.axis_index("s") * NC + lax.axis_index("c")
    pltpu.sync_copy(px_hbm, px_v)
    pltpu.sync_copy(py_hbm, py_v)
    pltpu.sync_copy(pz_hbm, pz_v)
    base0 = wid * EPW

    def blk(b, carry):
        base = base0 + b * EBG
        pltpu.sync_copy(src_hbm.at[pl.ds(base, EBG)], si_v)
        pltpu.sync_copy(dst_hbm.at[pl.ds(base, EBG)], di_v)

        def sub(j, c2):
            s16 = si_v[pl.ds(j * 16, 16)]
            d16 = di_v[pl.ds(j * 16, 16)]
            dx = plsc.load_gather(px_v, [s16]) - plsc.load_gather(px_v, [d16])
            dy = plsc.load_gather(py_v, [s16]) - plsc.load_gather(py_v, [d16])
            dz = plsc.load_gather(pz_v, [s16]) - plsc.load_gather(pz_v, [d16])
            r2 = jnp.maximum(dx * dx + dy * dy + dz * dz, 1e-24)
            ii = plsc.bitcast(r2, jnp.int32)
            ii = jnp.int32(0x5F3759DF) - lax.shift_right_logical(ii, 1)
            y = plsc.bitcast(ii, jnp.float32)
            y = y * (1.5 - 0.5 * r2 * y * y)
            y = y * (1.5 - 0.5 * r2 * y * y)
            y = y * (1.5 - 0.5 * r2 * y * y)
            u = (r2 * y) * INV_H + 0.5
            k = jnp.minimum(u.astype(jnp.int32), KTAB - 1)
            ko_v[pl.ds(j * 16, 16)] = k
            return c2

        lax.fori_loop(0, EBG // 16, sub, 0)
        pltpu.sync_copy(ko_v, ki_hbm.at[pl.ds(base, EBG)])
        return carry

    lax.fori_loop(0, EPW // EBG, blk, 0)


NBLK = EPW // EB
CH = 2000
CB = CH // EB
NCH = EPW // CH


@functools.partial(
    pl.kernel,
    out_type=jax.ShapeDtypeStruct((NC, N, D), jnp.float32),
    mesh=_mesh,
    scratch_types=[
        pltpu.VMEM((CH,), jnp.int32),
        pltpu.VMEM((CH,), jnp.int32),
        pltpu.VMEM((CB, EB), jnp.int32),
        pltpu.VMEM((EB, D), jnp.float32),
        pltpu.VMEM((EB, D), jnp.float32),
        pltpu.VMEM((EB, D), jnp.float32),
        pltpu.VMEM((EB, D), jnp.float32),
        pltpu.VMEM_SHARED((N, D), jnp.float32),
        pltpu.SemaphoreType.DMA,
        pltpu.SemaphoreType.DMA,
        pltpu.SemaphoreType.DMA,
        pltpu.SemaphoreType.DMA,
    ],
    compiler_params=_sc_params,
)
def _gather_mul_scatter(tab_hbm, hs_hbm, ki_hbm, src_hbm, dst2_hbm, zeros_hbm,
                        out_hbm, ki_c, si_c, di2, ta0, ta1, hg0, hg1,
                        agg_sh, st0, st1, sh0, sh1):
    c = lax.axis_index("c")
    s = lax.axis_index("s")
    wid = s * NC + c
    base0 = wid * EPW
    pltpu.sync_copy(zeros_hbm.at[pl.ds(s * NPT, NPT)],
                    agg_sh.at[pl.ds(s * NPT, NPT)])
    plsc.subcore_barrier()

    tas = (ta0, ta1)
    hgs = (hg0, hg1)
    sts = (st0, st1)
    shs = (sh0, sh1)

    def fire(blk, slot):
        off = blk * EB
        pltpu.async_copy(tab_hbm.at[ki_c.at[pl.ds(off, EB)]],
                         tas[slot], sts[slot])
        pltpu.async_copy(hs_hbm.at[si_c.at[pl.ds(off, EB)]],
                         hgs[slot], shs[slot])

    def drain_and_process(blk, slot):
        pltpu.make_async_copy(zeros_hbm.at[pl.ds(0, EB)],
                              tas[slot], sts[slot]).wait()
        pltpu.make_async_copy(zeros_hbm.at[pl.ds(0, EB)],
                              hgs[slot], shs[slot]).wait()

        ta_v = tas[slot]
        hg_v = hgs[slot]

        def row(rr, c2):
            for ch2 in range(D // 16):
                sl = pl.ds(ch2 * 16, 16)
                ta_v[rr, sl] = ta_v[rr, sl] * hg_v[rr, sl]
            return c2

        lax.fori_loop(0, EB, row, 0, unroll=2)
        pltpu.sync_copy(ta_v, agg_sh.at[di2.at[blk]], add=True)

    def chunk(ch, carry):
        cbase = base0 + ch * CH
        pltpu.sync_copy(ki_hbm.at[pl.ds(cbase, CH)], ki_c)
        pltpu.sync_copy(src_hbm.at[pl.ds(cbase, CH)], si_c)
        pltpu.sync_copy(dst2_hbm.at[pl.ds(wid * NBLK + ch * CB, CB)], di2)
        fire(0, 0)
        fire(1, 1)

        def g_loop(g, c2):
            for slot in range(2):
                blk = 2 * g + slot
                drain_and_process(blk, slot)
                nxt = blk + 2

                @pl.when(nxt < CB)
                def _():
                    fire(nxt, slot)
            return c2

        lax.fori_loop(0, CB // 2, g_loop, 0)
        if CB % 2 == 1:
            drain_and_process(CB - 1, 0)
        return carry

    lax.fori_loop(0, NCH, chunk, 0)
    plsc.subcore_barrier()
    pltpu.sync_copy(agg_sh.at[pl.ds(s * NPT, NPT)],
                    out_hbm.at[c, pl.ds(s * NPT, NPT)])


def _table_body(wr1_ref, wr2_ref, wr3_ref, t0_ref, t1_ref):
    i = pl.program_id(0)
    row0 = i * 512
    ridx = (lax.broadcasted_iota(jnp.int32, (512, 1), 0) + row0)
    r = ridx.astype(jnp.float32) * TAB_H
    rs = jnp.maximum(r, 1e-9)
    n = (lax.broadcasted_iota(jnp.int32, (1, NB), 1) + 1).astype(jnp.float32)
    rb = (np.float32(np.sqrt(2.0 / RMAX))
          * jnp.sin(rs * (np.pi / RMAX) * n) / rs)
    x = r * np.float32(1.0 / RMAX)
    x2 = x * x
    x5 = x2 * x2 * x
    env = (1.0 - 21.0 * x5 + 35.0 * x5 * x - 15.0 * x5 * x2)
    env = jnp.where(x < 1.0, env, 0.0)
    rb = rb * env
    outs = (t0_ref, t1_ref)
    for i2 in range(NI):
        t = rb @ wr1_ref[i2]
        t = t * jax.nn.sigmoid(t)
        t = t @ wr2_ref[i2]
        t = t * jax.nn.sigmoid(t)
        outs[i2][:, :] = t @ wr3_ref[i2]


def _table(Wr1, Wr2, Wr3):
    return pl.pallas_call(
        _table_body,
        grid=(KTAB // 512,),
        in_specs=[
            pl.BlockSpec((NI, NB, H1), lambda i: (0, 0, 0)),
            pl.BlockSpec((NI, H1, H1), lambda i: (0, 0, 0)),
            pl.BlockSpec((NI, H1, D), lambda i: (0, 0, 0)),
        ],
        out_specs=[
            pl.BlockSpec((512, D), lambda i: (i, 0)),
            pl.BlockSpec((512, D), lambda i: (i, 0)),
        ],
        out_shape=[
            jax.ShapeDtypeStruct((KTAB, D), jnp.float32),
            jax.ShapeDtypeStruct((KTAB, D), jnp.float32),
        ],
    )(Wr1, Wr2, Wr3)


def _node_init_body(sp_ref, wemb_ref, e0_ref, wup0_ref, h_ref, hs_ref, en_ref):
    sp = sp_ref[:, :]
    zi = lax.broadcasted_iota(jnp.int32, (1, Z), 1)
    oh = (sp == zi).astype(jnp.float32)
    h = oh @ wemb_ref[:, :]
    h_ref[:, :] = h
    hs_ref[:, :] = h @ wup0_ref[:, :]
    en_ref[:, :] = oh @ e0_ref[:, :]


def _node_init(species2, W_embed, E0c, Wup0, bn=2000):
    grid = N // bn
    return pl.pallas_call(
        _node_init_body,
        grid=(grid,),
        in_specs=[
            pl.BlockSpec((bn, 1), lambda i: (i, 0)),
            pl.BlockSpec((Z, D), lambda i: (0, 0)),
            pl.BlockSpec((Z, 1), lambda i: (0, 0)),
            pl.BlockSpec((D, D), lambda i: (0, 0)),
        ],
        out_specs=[
            pl.BlockSpec((bn, D), lambda i: (i, 0)),
            pl.BlockSpec((bn, D), lambda i: (i, 0)),
            pl.BlockSpec((bn, 1), lambda i: (i, 0)),
        ],
        out_shape=[
            jax.ShapeDtypeStruct((N, D), jnp.float32),
            jax.ShapeDtypeStruct((N, D), jnp.float32),
            jax.ShapeDtypeStruct((N, 1), jnp.float32),
        ],
    )(species2, W_embed, E0c, Wup0)


def _node_update_body(final, agg_ref, h_ref, sp_ref, en_ref, wout_ref,
                      wskip_ref, wprod_ref, wpl_ref, wro_ref, wup_ref,
                      wh_ref, wo_ref, h_out, hs_out, en_out):
    agg = (agg_ref[0] + agg_ref[1]) * np.float32(1.0 / AVG)
    h1 = agg @ wout_ref[:, :] + h_ref[:, :] @ wskip_ref[:, :]
    sp = sp_ref[:, :]
    zi = lax.broadcasted_iota(jnp.int32, (1, Z), 1)
    oh = (sp == zi).astype(jnp.float32)
    w = oh @ wprod_ref[:, :]
    g = (w[:, 0:D] * h1 + w[:, D:2 * D] * (h1 * h1)
         + w[:, 2 * D:3 * D] * (h1 * h1 * h1))
    h2 = g @ wpl_ref[:, :]
    h_out[:, :] = h2
    if final:
        t = h2 @ wh_ref[:, :]
        t = t * jax.nn.sigmoid(t)
        en_out[:, :] = en_ref[:, :] + t @ wo_ref[:, :]
        hs_out[:, :] = h2
    else:
        en_out[:, :] = en_ref[:, :] + h2 @ wro_ref[:, :]
        hs_out[:, :] = h2 @ wup_ref[:, :]


def _node_update(final, agg2, h, species2, en, Wout_i, Wskip_i, WprodF_i,
                 WprodLin_i, Wro_i, Wup_n, Wh, Wo, bn=2000):
    grid = N // bn
    return pl.pallas_call(
        functools.partial(_node_update_body, final),
        grid=(grid,),
        in_specs=[
            pl.BlockSpec((NC, bn, D), lambda i: (0, i, 0)),
            pl.BlockSpec((bn, D), lambda i: (i, 0)),
            pl.BlockSpec((bn, 1), lambda i: (i, 0)),
            pl.BlockSpec((bn, 1), lambda i: (i, 0)),
            pl.BlockSpec((D, D), lambda i: (0, 0)),
            pl.BlockSpec((D, D), lambda i: (0, 0)),
            pl.BlockSpec((Z, 3 * D), lambda i: (0, 0)),
            pl.BlockSpec((D, D), lambda i: (0, 0)),
            pl.BlockSpec((D, 1), lambda i: (0, 0)),
            pl.BlockSpec((D, D), lambda i: (0, 0)),
            pl.BlockSpec((D, MLPH), lambda i: (0, 0)),
            pl.BlockSpec((MLPH, 1), lambda i: (0, 0)),
        ],
        out_specs=[
            pl.BlockSpec((bn, D), lambda i: (i, 0)),
            pl.BlockSpec((bn, D), lambda i: (i, 0)),
            pl.BlockSpec((bn, 1), lambda i: (i, 0)),
        ],
        out_shape=[
            jax.ShapeDtypeStruct((N, D), jnp.float32),
            jax.ShapeDtypeStruct((N, D), jnp.float32),
            jax.ShapeDtypeStruct((N, 1), jnp.float32),
        ],
    )(agg2, h, species2, en, Wout_i, Wskip_i, WprodF_i, WprodLin_i,
      Wro_i, Wup_n, Wh, Wo)


def kernel(positions, species, edge_index, W_embed, E0, Wr1, Wr2, Wr3,
           Wup, Wout, Wskip, Wprod, WprodLin, Wro, Wh, Wo):
    src = edge_index[0]
    dst = edge_index[1]
    px = positions[:, 0]
    py = positions[:, 1]
    pz = positions[:, 2]
    species2 = species.reshape(N, 1).astype(jnp.int32)
    zeros_nd = jnp.zeros((N, D), jnp.float32)

    kidx = _geom(px, py, pz, src, dst)
    t0, t1 = _table(Wr1, Wr2, Wr3)
    h, hs, en = _node_init(species2, W_embed, E0.reshape(Z, 1), Wup[0])

    dst2 = dst.reshape(E // EB, EB)
    tabs = (t0, t1)
    for i in range(NI):
        agg2 = _gather_mul_scatter(tabs[i], hs, kidx, src, dst2, zeros_nd)
        h, hs, en = _node_update(
            i == NI - 1, agg2, h, species2, en,
            Wout[i], Wskip[i], Wprod[i].reshape(Z, 3 * D), WprodLin[i],
            Wro[i], Wup[(i + 1) % NI], Wh, Wo)
    return en.reshape(N)


# speedup vs baseline: 8.6131x; 1.8427x over previous
import functools

import jax
import jax.numpy as jnp
import numpy as np
from jax import lax
from jax.experimental import pallas as pl
from jax.experimental.pallas import tpu as pltpu
from jax.experimental.pallas import tpu_sc as plsc

N = 10000
E = 320000
D = 128
Z = 10
NB = 8
RMAX = 5.0
H1 = 64
MLPH = 16
NI = 2
AVG = 32.0

K = 16384
KTAB = 16896
TAB_H = np.float32(5.25 / K)
INV_H = np.float32(K / 5.25)

NC = 2
NS = 16
NW = NC * NS
EPW = E // NW
EBG = 2000
EB = 80
NPT = N // NS

_mesh = plsc.VectorSubcoreMesh(core_axis_name="c", subcore_axis_name="s")
_sc_params = pltpu.CompilerParams(needs_layout_passes=False,
                                  use_tc_tiling_on_sc=False)


@functools.partial(
    pl.kernel,
    out_type=jax.ShapeDtypeStruct((E,), jnp.int32),
    mesh=_mesh,
    scratch_types=[
        pltpu.VMEM((N,), jnp.float32),
        pltpu.VMEM((N,), jnp.float32),
        pltpu.VMEM((N,), jnp.float32),
        pltpu.VMEM((EBG,), jnp.int32),
        pltpu.VMEM((EBG,), jnp.int32),
        pltpu.VMEM((EBG,), jnp.int32),
    ],
    compiler_params=_sc_params,
)
def _geom(px_hbm, py_hbm, pz_hbm, src_hbm, dst_hbm, ki_hbm,
          px_v, py_v, pz_v, si_v, di_v, ko_v):
    wid = lax.axis_index("s") * NC + lax.axis_index("c")
    pltpu.sync_copy(px_hbm, px_v)
    pltpu.sync_copy(py_hbm, py_v)
    pltpu.sync_copy(pz_hbm, pz_v)
    base0 = wid * EPW

    def blk(b, carry):
        base = base0 + b * EBG
        pltpu.sync_copy(src_hbm.at[pl.ds(base, EBG)], si_v)
        pltpu.sync_copy(dst_hbm.at[pl.ds(base, EBG)], di_v)

        def sub(j, c2):
            s16 = si_v[pl.ds(j * 16, 16)]
            d16 = di_v[pl.ds(j * 16, 16)]
            dx = plsc.load_gather(px_v, [s16]) - plsc.load_gather(px_v, [d16])
            dy = plsc.load_gather(py_v, [s16]) - plsc.load_gather(py_v, [d16])
            dz = plsc.load_gather(pz_v, [s16]) - plsc.load_gather(pz_v, [d16])
            r2 = jnp.maximum(dx * dx + dy * dy + dz * dz, 1e-24)
            ii = plsc.bitcast(r2, jnp.int32)
            ii = jnp.int32(0x5F3759DF) - lax.shift_right_logical(ii, 1)
            y = plsc.bitcast(ii, jnp.float32)
            y = y * (1.5 - 0.5 * r2 * y * y)
            y = y * (1.5 - 0.5 * r2 * y * y)
            y = y * (1.5 - 0.5 * r2 * y * y)
            u = (r2 * y) * INV_H + 0.5
            k = jnp.minimum(u.astype(jnp.int32), KTAB - 1)
            ko_v[pl.ds(j * 16, 16)] = k
            return c2

        lax.fori_loop(0, EBG // 16, sub, 0)
        pltpu.sync_copy(ko_v, ki_hbm.at[pl.ds(base, EBG)])
        return carry

    lax.fori_loop(0, EPW // EBG, blk, 0)


NBLK = EPW // EB
CH = 2000
CB = CH // EB
NCH = EPW // CH


@functools.partial(
    pl.kernel,
    out_type=jax.ShapeDtypeStruct((NC, N, D), jnp.float32),
    mesh=_mesh,
    scratch_types=[
        pltpu.VMEM((CH,), jnp.int32),
        pltpu.VMEM((CH,), jnp.int32),
        pltpu.VMEM((CB, EB), jnp.int32),
        pltpu.VMEM((EB, D), jnp.float32),
        pltpu.VMEM((EB, D), jnp.float32),
        pltpu.VMEM((EB, D), jnp.float32),
        pltpu.VMEM((EB, D), jnp.float32),
        pltpu.VMEM_SHARED((N, D), jnp.float32),
        pltpu.SemaphoreType.DMA,
        pltpu.SemaphoreType.DMA,
        pltpu.SemaphoreType.DMA,
        pltpu.SemaphoreType.DMA,
    ],
    compiler_params=_sc_params,
)
def _gather_mul_scatter(tab_hbm, hs_hbm, ki_hbm, src_hbm, dst2_hbm, zeros_hbm,
                        out_hbm, ki_c, si_c, di2, ta0, ta1, hg0, hg1,
                        agg_sh, st0, st1, sh0, sh1):
    c = lax.axis_index("c")
    s = lax.axis_index("s")
    wid = s * NC + c
    base0 = wid * EPW
    pltpu.sync_copy(zeros_hbm.at[pl.ds(s * NPT, NPT)],
                    agg_sh.at[pl.ds(s * NPT, NPT)])
    plsc.subcore_barrier()

    tas = (ta0, ta1)
    hgs = (hg0, hg1)
    sts = (st0, st1)
    shs = (sh0, sh1)

    def fire(blk, slot):
        off = blk * EB
        pltpu.async_copy(tab_hbm.at[ki_c.at[pl.ds(off, EB)]],
                         tas[slot], sts[slot])
        pltpu.async_copy(hs_hbm.at[si_c.at[pl.ds(off, EB)]],
                         hgs[slot], shs[slot])

    def drain_and_process(blk, slot):
        pltpu.make_async_copy(zeros_hbm.at[pl.ds(0, EB)],
                              tas[slot], sts[slot]).wait()
        pltpu.make_async_copy(zeros_hbm.at[pl.ds(0, EB)],
                              hgs[slot], shs[slot]).wait()

        ta_v = tas[slot]
        hg_v = hgs[slot]

        @plsc.parallel_loop(0, EB, 1, unroll=4)
        def _mul(rr):
            for ch2 in range(D // 16):
                sl = pl.ds(ch2 * 16, 16)
                ta_v[rr, sl] = ta_v[rr, sl] * hg_v[rr, sl]

        pltpu.sync_copy(ta_v, agg_sh.at[di2.at[blk]], add=True)

    def chunk(ch, carry):
        cbase = base0 + ch * CH
        pltpu.sync_copy(ki_hbm.at[pl.ds(cbase, CH)], ki_c)
        pltpu.sync_copy(src_hbm.at[pl.ds(cbase, CH)], si_c)
        pltpu.sync_copy(dst2_hbm.at[pl.ds(wid * NBLK + ch * CB, CB)], di2)
        fire(0, 0)
        fire(1, 1)

        def g_loop(g, c2):
            for slot in range(2):
                blk = 2 * g + slot
                drain_and_process(blk, slot)
                nxt = blk + 2

                @pl.when(nxt < CB)
                def _():
                    fire(nxt, slot)
            return c2

        lax.fori_loop(0, CB // 2, g_loop, 0)
        if CB % 2 == 1:
            drain_and_process(CB - 1, 0)
        return carry

    lax.fori_loop(0, NCH, chunk, 0)
    plsc.subcore_barrier()
    pltpu.sync_copy(agg_sh.at[pl.ds(s * NPT, NPT)],
                    out_hbm.at[c, pl.ds(s * NPT, NPT)])


def _table_body(wr1_ref, wr2_ref, wr3_ref, t0_ref, t1_ref):
    i = pl.program_id(0)
    row0 = i * 512
    ridx = (lax.broadcasted_iota(jnp.int32, (512, 1), 0) + row0)
    r = ridx.astype(jnp.float32) * TAB_H
    rs = jnp.maximum(r, 1e-9)
    n = (lax.broadcasted_iota(jnp.int32, (1, NB), 1) + 1).astype(jnp.float32)
    rb = (np.float32(np.sqrt(2.0 / RMAX))
          * jnp.sin(rs * (np.pi / RMAX) * n) / rs)
    x = r * np.float32(1.0 / RMAX)
    x2 = x * x
    x5 = x2 * x2 * x
    env = (1.0 - 21.0 * x5 + 35.0 * x5 * x - 15.0 * x5 * x2)
    env = jnp.where(x < 1.0, env, 0.0)
    rb = rb * env
    outs = (t0_ref, t1_ref)
    for i2 in range(NI):
        t = rb @ wr1_ref[i2]
        t = t * jax.nn.sigmoid(t)
        t = t @ wr2_ref[i2]
        t = t * jax.nn.sigmoid(t)
        outs[i2][:, :] = t @ wr3_ref[i2]


def _table(Wr1, Wr2, Wr3):
    return pl.pallas_call(
        _table_body,
        grid=(KTAB // 512,),
        in_specs=[
            pl.BlockSpec((NI, NB, H1), lambda i: (0, 0, 0)),
            pl.BlockSpec((NI, H1, H1), lambda i: (0, 0, 0)),
            pl.BlockSpec((NI, H1, D), lambda i: (0, 0, 0)),
        ],
        out_specs=[
            pl.BlockSpec((512, D), lambda i: (i, 0)),
            pl.BlockSpec((512, D), lambda i: (i, 0)),
        ],
        out_shape=[
            jax.ShapeDtypeStruct((KTAB, D), jnp.float32),
            jax.ShapeDtypeStruct((KTAB, D), jnp.float32),
        ],
    )(Wr1, Wr2, Wr3)


def _node_init_body(sp_ref, wemb_ref, e0_ref, wup0_ref, h_ref, hs_ref, en_ref):
    sp = sp_ref[:, :]
    zi = lax.broadcasted_iota(jnp.int32, (1, Z), 1)
    oh = (sp == zi).astype(jnp.float32)
    h = oh @ wemb_ref[:, :]
    h_ref[:, :] = h
    hs_ref[:, :] = h @ wup0_ref[:, :]
    en_ref[:, :] = oh @ e0_ref[:, :]


def _node_init(species2, W_embed, E0c, Wup0, bn=2000):
    grid = N // bn
    return pl.pallas_call(
        _node_init_body,
        grid=(grid,),
        in_specs=[
            pl.BlockSpec((bn, 1), lambda i: (i, 0)),
            pl.BlockSpec((Z, D), lambda i: (0, 0)),
            pl.BlockSpec((Z, 1), lambda i: (0, 0)),
            pl.BlockSpec((D, D), lambda i: (0, 0)),
        ],
        out_specs=[
            pl.BlockSpec((bn, D), lambda i: (i, 0)),
            pl.BlockSpec((bn, D), lambda i: (i, 0)),
            pl.BlockSpec((bn, 1), lambda i: (i, 0)),
        ],
        out_shape=[
            jax.ShapeDtypeStruct((N, D), jnp.float32),
            jax.ShapeDtypeStruct((N, D), jnp.float32),
            jax.ShapeDtypeStruct((N, 1), jnp.float32),
        ],
    )(species2, W_embed, E0c, Wup0)


def _node_update_body(final, agg_ref, h_ref, sp_ref, en_ref, wout_ref,
                      wskip_ref, wprod_ref, wpl_ref, wro_ref, wup_ref,
                      wh_ref, wo_ref, h_out, hs_out, en_out):
    agg = (agg_ref[0] + agg_ref[1]) * np.float32(1.0 / AVG)
    h1 = agg @ wout_ref[:, :] + h_ref[:, :] @ wskip_ref[:, :]
    sp = sp_ref[:, :]
    zi = lax.broadcasted_iota(jnp.int32, (1, Z), 1)
    oh = (sp == zi).astype(jnp.float32)
    w = oh @ wprod_ref[:, :]
    g = (w[:, 0:D] * h1 + w[:, D:2 * D] * (h1 * h1)
         + w[:, 2 * D:3 * D] * (h1 * h1 * h1))
    h2 = g @ wpl_ref[:, :]
    h_out[:, :] = h2
    if final:
        t = h2 @ wh_ref[:, :]
        t = t * jax.nn.sigmoid(t)
        en_out[:, :] = en_ref[:, :] + t @ wo_ref[:, :]
        hs_out[:, :] = h2
    else:
        en_out[:, :] = en_ref[:, :] + h2 @ wro_ref[:, :]
        hs_out[:, :] = h2 @ wup_ref[:, :]


def _node_update(final, agg2, h, species2, en, Wout_i, Wskip_i, WprodF_i,
                 WprodLin_i, Wro_i, Wup_n, Wh, Wo, bn=2000):
    grid = N // bn
    return pl.pallas_call(
        functools.partial(_node_update_body, final),
        grid=(grid,),
        in_specs=[
            pl.BlockSpec((NC, bn, D), lambda i: (0, i, 0)),
            pl.BlockSpec((bn, D), lambda i: (i, 0)),
            pl.BlockSpec((bn, 1), lambda i: (i, 0)),
            pl.BlockSpec((bn, 1), lambda i: (i, 0)),
            pl.BlockSpec((D, D), lambda i: (0, 0)),
            pl.BlockSpec((D, D), lambda i: (0, 0)),
            pl.BlockSpec((Z, 3 * D), lambda i: (0, 0)),
            pl.BlockSpec((D, D), lambda i: (0, 0)),
            pl.BlockSpec((D, 1), lambda i: (0, 0)),
            pl.BlockSpec((D, D), lambda i: (0, 0)),
            pl.BlockSpec((D, MLPH), lambda i: (0, 0)),
            pl.BlockSpec((MLPH, 1), lambda i: (0, 0)),
        ],
        out_specs=[
            pl.BlockSpec((bn, D), lambda i: (i, 0)),
            pl.BlockSpec((bn, D), lambda i: (i, 0)),
            pl.BlockSpec((bn, 1), lambda i: (i, 0)),
        ],
        out_shape=[
            jax.ShapeDtypeStruct((N, D), jnp.float32),
            jax.ShapeDtypeStruct((N, D), jnp.float32),
            jax.ShapeDtypeStruct((N, 1), jnp.float32),
        ],
    )(agg2, h, species2, en, Wout_i, Wskip_i, WprodF_i, WprodLin_i,
      Wro_i, Wup_n, Wh, Wo)


def kernel(positions, species, edge_index, W_embed, E0, Wr1, Wr2, Wr3,
           Wup, Wout, Wskip, Wprod, WprodLin, Wro, Wh, Wo):
    src = edge_index[0]
    dst = edge_index[1]
    px = positions[:, 0]
    py = positions[:, 1]
    pz = positions[:, 2]
    species2 = species.reshape(N, 1).astype(jnp.int32)
    zeros_nd = jnp.zeros((N, D), jnp.float32)

    kidx = _geom(px, py, pz, src, dst)
    t0, t1 = _table(Wr1, Wr2, Wr3)
    h, hs, en = _node_init(species2, W_embed, E0.reshape(Z, 1), Wup[0])

    dst2 = dst.reshape(E // EB, EB)
    tabs = (t0, t1)
    for i in range(NI):
        agg2 = _gather_mul_scatter(tabs[i], hs, kidx, src, dst2, zeros_nd)
        h, hs, en = _node_update(
            i == NI - 1, agg2, h, species2, en,
            Wout[i], Wskip[i], Wprod[i].reshape(Z, 3 * D), WprodLin[i],
            Wro[i], Wup[(i + 1) % NI], Wh, Wo)
    return en.reshape(N)


# speedup vs baseline: 9.0367x; 1.0492x over previous
import functools

import jax
import jax.numpy as jnp
import numpy as np
from jax import lax
from jax.experimental import pallas as pl
from jax.experimental.pallas import tpu as pltpu
from jax.experimental.pallas import tpu_sc as plsc

N = 10000
E = 320000
D = 128
Z = 10
NB = 8
RMAX = 5.0
H1 = 64
MLPH = 16
NI = 2
AVG = 32.0

K = 8192
KTAB = 8704
TAB_H = np.float32(5.25 / K)
INV_H = np.float32(K / 5.25)

NC = 2
NS = 16
NW = NC * NS
EPW = E // NW
EBG = 2000
EB = 40
NPT = N // NS

_mesh = plsc.VectorSubcoreMesh(core_axis_name="c", subcore_axis_name="s")
_sc_params = pltpu.CompilerParams(needs_layout_passes=False,
                                  use_tc_tiling_on_sc=False)


@functools.partial(
    pl.kernel,
    out_type=jax.ShapeDtypeStruct((E,), jnp.int32),
    mesh=_mesh,
    scratch_types=[
        pltpu.VMEM((N,), jnp.float32),
        pltpu.VMEM((N,), jnp.float32),
        pltpu.VMEM((N,), jnp.float32),
        pltpu.VMEM((EBG,), jnp.int32),
        pltpu.VMEM((EBG,), jnp.int32),
        pltpu.VMEM((EBG,), jnp.int32),
    ],
    compiler_params=_sc_params,
)
def _geom(px_hbm, py_hbm, pz_hbm, src_hbm, dst_hbm, ki_hbm,
          px_v, py_v, pz_v, si_v, di_v, ko_v):
    wid = lax.axis_index("s") * NC + lax.axis_index("c")
    pltpu.sync_copy(px_hbm, px_v)
    pltpu.sync_copy(py_hbm, py_v)
    pltpu.sync_copy(pz_hbm, pz_v)
    base0 = wid * EPW

    def blk(b, carry):
        base = base0 + b * EBG
        pltpu.sync_copy(src_hbm.at[pl.ds(base, EBG)], si_v)
        pltpu.sync_copy(dst_hbm.at[pl.ds(base, EBG)], di_v)

        def sub(j, c2):
            s16 = si_v[pl.ds(j * 16, 16)]
            d16 = di_v[pl.ds(j * 16, 16)]
            dx = plsc.load_gather(px_v, [s16]) - plsc.load_gather(px_v, [d16])
            dy = plsc.load_gather(py_v, [s16]) - plsc.load_gather(py_v, [d16])
            dz = plsc.load_gather(pz_v, [s16]) - plsc.load_gather(pz_v, [d16])
            r2 = jnp.maximum(dx * dx + dy * dy + dz * dz, 1e-24)
            ii = plsc.bitcast(r2, jnp.int32)
            ii = jnp.int32(0x5F3759DF) - lax.shift_right_logical(ii, 1)
            y = plsc.bitcast(ii, jnp.float32)
            y = y * (1.5 - 0.5 * r2 * y * y)
            y = y * (1.5 - 0.5 * r2 * y * y)
            y = y * (1.5 - 0.5 * r2 * y * y)
            u = (r2 * y) * INV_H + 0.5
            k = jnp.minimum(u.astype(jnp.int32), KTAB - 1)
            ko_v[pl.ds(j * 16, 16)] = k
            return c2

        lax.fori_loop(0, EBG // 16, sub, 0)
        pltpu.sync_copy(ko_v, ki_hbm.at[pl.ds(base, EBG)])
        return carry

    lax.fori_loop(0, EPW // EBG, blk, 0)


NBLK = EPW // EB
CH = 2000
CB = CH // EB
NCH = EPW // CH


@functools.partial(
    pl.kernel,
    out_type=jax.ShapeDtypeStruct((NC, N, D), jnp.float32),
    mesh=_mesh,
    scratch_types=[
        pltpu.VMEM((CH,), jnp.int32),
        pltpu.VMEM((CH,), jnp.int32),
        pltpu.VMEM((CB, EB), jnp.int32),
        pltpu.VMEM((EB, D), jnp.float32),
        pltpu.VMEM((EB, D), jnp.float32),
        pltpu.VMEM((EB, D), jnp.float32),
        pltpu.VMEM((EB, D), jnp.float32),
        pltpu.VMEM((EB, D), jnp.float32),
        pltpu.VMEM((EB, D), jnp.float32),
        pltpu.VMEM_SHARED((N, D), jnp.float32),
        pltpu.SemaphoreType.DMA,
        pltpu.SemaphoreType.DMA,
        pltpu.SemaphoreType.DMA,
        pltpu.SemaphoreType.DMA,
        pltpu.SemaphoreType.DMA,
        pltpu.SemaphoreType.DMA,
    ],
    compiler_params=_sc_params,
)
def _gather_mul_scatter(tab_hbm, hs_hbm, ki_hbm, src_hbm, dst2_hbm, zeros_hbm,
                        out_hbm, ki_c, si_c, di2, ta0, ta1, hg0, hg1, m0, m1,
                        agg_sh, st0, st1, sh0, sh1, ss0, ss1):
    c = lax.axis_index("c")
    s = lax.axis_index("s")
    wid = s * NC + c
    base0 = wid * EPW
    pltpu.sync_copy(zeros_hbm.at[pl.ds(s * NPT, NPT)],
                    agg_sh.at[pl.ds(s * NPT, NPT)])
    plsc.subcore_barrier()

    tas = (ta0, ta1)
    hgs = (hg0, hg1)
    ms = (m0, m1)
    sts = (st0, st1)
    shs = (sh0, sh1)
    sss = (ss0, ss1)

    def fire(blk, slot):
        off = blk * EB
        pltpu.async_copy(tab_hbm.at[ki_c.at[pl.ds(off, EB)]],
                         tas[slot], sts[slot])
        pltpu.async_copy(hs_hbm.at[si_c.at[pl.ds(off, EB)]],
                         hgs[slot], shs[slot])

    def drain_and_process(blk, gblk, slot):
        pltpu.make_async_copy(zeros_hbm.at[pl.ds(0, EB)],
                              tas[slot], sts[slot]).wait()
        pltpu.make_async_copy(zeros_hbm.at[pl.ds(0, EB)],
                              hgs[slot], shs[slot]).wait()

        @pl.when(gblk >= 2)
        def _():
            pltpu.make_async_copy(zeros_hbm.at[pl.ds(0, EB)],
                                  ms[slot], sss[slot]).wait()

        ta_v = tas[slot]
        hg_v = hgs[slot]
        m_v = ms[slot]

        @plsc.parallel_loop(0, EB, 1, unroll=4)
        def _mul(rr):
            for ch2 in range(D // 16):
                sl = pl.ds(ch2 * 16, 16)
                m_v[rr, sl] = ta_v[rr, sl] * hg_v[rr, sl]

        nxt = blk + 2

        @pl.when(nxt < CB)
        def _():
            fire(nxt, slot)

        pltpu.async_copy(m_v, agg_sh.at[di2.at[blk]], sss[slot], add=True)

    def chunk(ch, carry):
        cbase = base0 + ch * CH
        pltpu.sync_copy(ki_hbm.at[pl.ds(cbase, CH)], ki_c)
        pltpu.sync_copy(src_hbm.at[pl.ds(cbase, CH)], si_c)
        pltpu.sync_copy(dst2_hbm.at[pl.ds(wid * NBLK + ch * CB, CB)], di2)
        fire(0, 0)
        fire(1, 1)

        def g_loop(g, c2):
            for slot in range(2):
                blk = 2 * g + slot
                drain_and_process(blk, ch * CB + blk, slot)
            return c2

        lax.fori_loop(0, CB // 2, g_loop, 0)
        if CB % 2 == 1:
            drain_and_process(CB - 1, ch * CB + CB - 1, 0)
        return carry

    lax.fori_loop(0, NCH, chunk, 0)
    pltpu.make_async_copy(zeros_hbm.at[pl.ds(0, EB)], m0, ss0).wait()
    pltpu.make_async_copy(zeros_hbm.at[pl.ds(0, EB)], m1, ss1).wait()
    plsc.subcore_barrier()
    pltpu.sync_copy(agg_sh.at[pl.ds(s * NPT, NPT)],
                    out_hbm.at[c, pl.ds(s * NPT, NPT)])


def _table_body(wr1_ref, wr2_ref, wr3_ref, t0_ref, t1_ref):
    i = pl.program_id(0)
    row0 = i * 512
    ridx = (lax.broadcasted_iota(jnp.int32, (512, 1), 0) + row0)
    r = ridx.astype(jnp.float32) * TAB_H
    rs = jnp.maximum(r, 1e-9)
    n = (lax.broadcasted_iota(jnp.int32, (1, NB), 1) + 1).astype(jnp.float32)
    rb = (np.float32(np.sqrt(2.0 / RMAX))
          * jnp.sin(rs * (np.pi / RMAX) * n) / rs)
    x = r * np.float32(1.0 / RMAX)
    x2 = x * x
    x5 = x2 * x2 * x
    env = (1.0 - 21.0 * x5 + 35.0 * x5 * x - 15.0 * x5 * x2)
    env = jnp.where(x < 1.0, env, 0.0)
    rb = rb * env
    outs = (t0_ref, t1_ref)
    for i2 in range(NI):
        t = rb @ wr1_ref[i2]
        t = t * jax.nn.sigmoid(t)
        t = t @ wr2_ref[i2]
        t = t * jax.nn.sigmoid(t)
        outs[i2][:, :] = t @ wr3_ref[i2]


def _table(Wr1, Wr2, Wr3):
    return pl.pallas_call(
        _table_body,
        grid=(KTAB // 512,),
        in_specs=[
            pl.BlockSpec((NI, NB, H1), lambda i: (0, 0, 0)),
            pl.BlockSpec((NI, H1, H1), lambda i: (0, 0, 0)),
            pl.BlockSpec((NI, H1, D), lambda i: (0, 0, 0)),
        ],
        out_specs=[
            pl.BlockSpec((512, D), lambda i: (i, 0)),
            pl.BlockSpec((512, D), lambda i: (i, 0)),
        ],
        out_shape=[
            jax.ShapeDtypeStruct((KTAB, D), jnp.float32),
            jax.ShapeDtypeStruct((KTAB, D), jnp.float32),
        ],
    )(Wr1, Wr2, Wr3)


def _node_init_body(sp_ref, wemb_ref, e0_ref, wup0_ref, h_ref, hs_ref, en_ref):
    sp = sp_ref[:, :]
    zi = lax.broadcasted_iota(jnp.int32, (1, Z), 1)
    oh = (sp == zi).astype(jnp.float32)
    h = oh @ wemb_ref[:, :]
    h_ref[:, :] = h
    hs_ref[:, :] = h @ wup0_ref[:, :]
    en_ref[:, :] = oh @ e0_ref[:, :]


def _node_init(species2, W_embed, E0c, Wup0, bn=2000):
    grid = N // bn
    return pl.pallas_call(
        _node_init_body,
        grid=(grid,),
        in_specs=[
            pl.BlockSpec((bn, 1), lambda i: (i, 0)),
            pl.BlockSpec((Z, D), lambda i: (0, 0)),
            pl.BlockSpec((Z, 1), lambda i: (0, 0)),
            pl.BlockSpec((D, D), lambda i: (0, 0)),
        ],
        out_specs=[
            pl.BlockSpec((bn, D), lambda i: (i, 0)),
            pl.BlockSpec((bn, D), lambda i: (i, 0)),
            pl.BlockSpec((bn, 1), lambda i: (i, 0)),
        ],
        out_shape=[
            jax.ShapeDtypeStruct((N, D), jnp.float32),
            jax.ShapeDtypeStruct((N, D), jnp.float32),
            jax.ShapeDtypeStruct((N, 1), jnp.float32),
        ],
    )(species2, W_embed, E0c, Wup0)


def _node_update_body(final, agg_ref, h_ref, sp_ref, en_ref, wout_ref,
                      wskip_ref, wprod_ref, wpl_ref, wro_ref, wup_ref,
                      wh_ref, wo_ref, h_out, hs_out, en_out):
    agg = (agg_ref[0] + agg_ref[1]) * np.float32(1.0 / AVG)
    h1 = agg @ wout_ref[:, :] + h_ref[:, :] @ wskip_ref[:, :]
    sp = sp_ref[:, :]
    zi = lax.broadcasted_iota(jnp.int32, (1, Z), 1)
    oh = (sp == zi).astype(jnp.float32)
    w = oh @ wprod_ref[:, :]
    g = (w[:, 0:D] * h1 + w[:, D:2 * D] * (h1 * h1)
         + w[:, 2 * D:3 * D] * (h1 * h1 * h1))
    h2 = g @ wpl_ref[:, :]
    h_out[:, :] = h2
    if final:
        t = h2 @ wh_ref[:, :]
        t = t * jax.nn.sigmoid(t)
        en_out[:, :] = en_ref[:, :] + t @ wo_ref[:, :]
        hs_out[:, :] = h2
    else:
        en_out[:, :] = en_ref[:, :] + h2 @ wro_ref[:, :]
        hs_out[:, :] = h2 @ wup_ref[:, :]


def _node_update(final, agg2, h, species2, en, Wout_i, Wskip_i, WprodF_i,
                 WprodLin_i, Wro_i, Wup_n, Wh, Wo, bn=2000):
    grid = N // bn
    return pl.pallas_call(
        functools.partial(_node_update_body, final),
        grid=(grid,),
        in_specs=[
            pl.BlockSpec((NC, bn, D), lambda i: (0, i, 0)),
            pl.BlockSpec((bn, D), lambda i: (i, 0)),
            pl.BlockSpec((bn, 1), lambda i: (i, 0)),
            pl.BlockSpec((bn, 1), lambda i: (i, 0)),
            pl.BlockSpec((D, D), lambda i: (0, 0)),
            pl.BlockSpec((D, D), lambda i: (0, 0)),
            pl.BlockSpec((Z, 3 * D), lambda i: (0, 0)),
            pl.BlockSpec((D, D), lambda i: (0, 0)),
            pl.BlockSpec((D, 1), lambda i: (0, 0)),
            pl.BlockSpec((D, D), lambda i: (0, 0)),
            pl.BlockSpec((D, MLPH), lambda i: (0, 0)),
            pl.BlockSpec((MLPH, 1), lambda i: (0, 0)),
        ],
        out_specs=[
            pl.BlockSpec((bn, D), lambda i: (i, 0)),
            pl.BlockSpec((bn, D), lambda i: (i, 0)),
            pl.BlockSpec((bn, 1), lambda i: (i, 0)),
        ],
        out_shape=[
            jax.ShapeDtypeStruct((N, D), jnp.float32),
            jax.ShapeDtypeStruct((N, D), jnp.float32),
            jax.ShapeDtypeStruct((N, 1), jnp.float32),
        ],
    )(agg2, h, species2, en, Wout_i, Wskip_i, WprodF_i, WprodLin_i,
      Wro_i, Wup_n, Wh, Wo)


def kernel(positions, species, edge_index, W_embed, E0, Wr1, Wr2, Wr3,
           Wup, Wout, Wskip, Wprod, WprodLin, Wro, Wh, Wo):
    src = edge_index[0]
    dst = edge_index[1]
    px = positions[:, 0]
    py = positions[:, 1]
    pz = positions[:, 2]
    species2 = species.reshape(N, 1).astype(jnp.int32)
    zeros_nd = jnp.zeros((N, D), jnp.float32)

    kidx = _geom(px, py, pz, src, dst)
    t0, t1 = _table(Wr1, Wr2, Wr3)
    h, hs, en = _node_init(species2, W_embed, E0.reshape(Z, 1), Wup[0])

    dst2 = dst.reshape(E // EB, EB)
    tabs = (t0, t1)
    for i in range(NI):
        agg2 = _gather_mul_scatter(tabs[i], hs, kidx, src, dst2, zeros_nd)
        h, hs, en = _node_update(
            i == NI - 1, agg2, h, species2, en,
            Wout[i], Wskip[i], Wprod[i].reshape(Z, 3 * D), WprodLin[i],
            Wro[i], Wup[(i + 1) % NI], Wh, Wo)
    return en.reshape(N)


# speedup vs baseline: 10.8508x; 1.2008x over previous
import functools

import jax
import jax.numpy as jnp
import numpy as np
from jax import lax
from jax.experimental import pallas as pl
from jax.experimental.pallas import tpu as pltpu
from jax.experimental.pallas import tpu_sc as plsc

N = 10000
E = 320000
D = 128
Z = 10
NB = 8
RMAX = 5.0
H1 = 64
MLPH = 16
NI = 2
AVG = 32.0

K = 8192
KTAB = 8704
TAB_H = np.float32(5.25 / K)
INV_H = np.float32(K / 5.25)

_O = np.arange(D) % 32
PRM = np.asarray(32 * (np.arange(D) // 32) + (_O % 2) * 16 + _O // 2,
                 dtype=np.int32)

NC = 2
NS = 16
NW = NC * NS
EPW = E // NW
EBG = 2000
EB = 40
NPT = N // NS

_mesh = plsc.VectorSubcoreMesh(core_axis_name="c", subcore_axis_name="s")
_sc_params = pltpu.CompilerParams(needs_layout_passes=False,
                                  use_tc_tiling_on_sc=False)


@functools.partial(
    pl.kernel,
    out_type=jax.ShapeDtypeStruct((E,), jnp.int32),
    mesh=_mesh,
    scratch_types=[
        pltpu.VMEM((N,), jnp.float32),
        pltpu.VMEM((N,), jnp.float32),
        pltpu.VMEM((N,), jnp.float32),
        pltpu.VMEM((EBG,), jnp.int32),
        pltpu.VMEM((EBG,), jnp.int32),
        pltpu.VMEM((EBG,), jnp.int32),
    ],
    compiler_params=_sc_params,
)
def _geom(px_hbm, py_hbm, pz_hbm, src_hbm, dst_hbm, ki_hbm,
          px_v, py_v, pz_v, si_v, di_v, ko_v):
    wid = lax.axis_index("s") * NC + lax.axis_index("c")
    pltpu.sync_copy(px_hbm, px_v)
    pltpu.sync_copy(py_hbm, py_v)
    pltpu.sync_copy(pz_hbm, pz_v)
    base0 = wid * EPW

    def blk(b, carry):
        base = base0 + b * EBG
        pltpu.sync_copy(src_hbm.at[pl.ds(base, EBG)], si_v)
        pltpu.sync_copy(dst_hbm.at[pl.ds(base, EBG)], di_v)

        def sub(j, c2):
            s16 = si_v[pl.ds(j * 16, 16)]
            d16 = di_v[pl.ds(j * 16, 16)]
            dx = plsc.load_gather(px_v, [s16]) - plsc.load_gather(px_v, [d16])
            dy = plsc.load_gather(py_v, [s16]) - plsc.load_gather(py_v, [d16])
            dz = plsc.load_gather(pz_v, [s16]) - plsc.load_gather(pz_v, [d16])
            r2 = jnp.maximum(dx * dx + dy * dy + dz * dz, 1e-24)
            ii = plsc.bitcast(r2, jnp.int32)
            ii = jnp.int32(0x5F3759DF) - lax.shift_right_logical(ii, 1)
            y = plsc.bitcast(ii, jnp.float32)
            y = y * (1.5 - 0.5 * r2 * y * y)
            y = y * (1.5 - 0.5 * r2 * y * y)
            y = y * (1.5 - 0.5 * r2 * y * y)
            u = (r2 * y) * INV_H + 0.5
            k = jnp.minimum(u.astype(jnp.int32), KTAB - 1)
            ko_v[pl.ds(j * 16, 16)] = k
            return c2

        lax.fori_loop(0, EBG // 16, sub, 0)
        pltpu.sync_copy(ko_v, ki_hbm.at[pl.ds(base, EBG)])
        return carry

    lax.fori_loop(0, EPW // EBG, blk, 0)


NBLK = EPW // EB
CH = 2000
CB = CH // EB
NCH = EPW // CH


@functools.partial(
    pl.kernel,
    out_type=jax.ShapeDtypeStruct((NC, N, D), jnp.float32),
    mesh=_mesh,
    scratch_types=[
        pltpu.VMEM((CH,), jnp.int32),
        pltpu.VMEM((CH,), jnp.int32),
        pltpu.VMEM((CB, EB), jnp.int32),
        pltpu.VMEM((EB, D), jnp.bfloat16),
        pltpu.VMEM((EB, D), jnp.bfloat16),
        pltpu.VMEM((EB, D), jnp.bfloat16),
        pltpu.VMEM((EB, D), jnp.bfloat16),
        pltpu.VMEM((EB, D), jnp.float32),
        pltpu.VMEM((EB, D), jnp.float32),
        pltpu.VMEM_SHARED((N, D), jnp.float32),
        pltpu.SemaphoreType.DMA,
        pltpu.SemaphoreType.DMA,
        pltpu.SemaphoreType.DMA,
        pltpu.SemaphoreType.DMA,
        pltpu.SemaphoreType.DMA,
        pltpu.SemaphoreType.DMA,
    ],
    compiler_params=_sc_params,
)
def _gather_mul_scatter(tab_hbm, hs_hbm, ki_hbm, src_hbm, dst2_hbm, zeros_hbm,
                        out_hbm, ki_c, si_c, di2, ta0, ta1, hg0, hg1, m0, m1,
                        agg_sh, st0, st1, sh0, sh1, ss0, ss1):
    c = lax.axis_index("c")
    s = lax.axis_index("s")
    wid = s * NC + c
    base0 = wid * EPW
    pltpu.sync_copy(zeros_hbm.at[pl.ds(s * NPT, NPT)],
                    agg_sh.at[pl.ds(s * NPT, NPT)])
    plsc.subcore_barrier()

    tas = (ta0, ta1)
    hgs = (hg0, hg1)
    ms = (m0, m1)
    sts = (st0, st1)
    shs = (sh0, sh1)
    sss = (ss0, ss1)

    def fire(blk, slot):
        off = blk * EB
        pltpu.async_copy(tab_hbm.at[ki_c.at[pl.ds(off, EB)]],
                         tas[slot], sts[slot])
        pltpu.async_copy(hs_hbm.at[si_c.at[pl.ds(off, EB)]],
                         hgs[slot], shs[slot])

    def drain_and_process(blk, gblk, slot):
        pltpu.make_async_copy(tab_hbm.at[pl.ds(0, EB)],
                              tas[slot], sts[slot]).wait()
        pltpu.make_async_copy(hs_hbm.at[pl.ds(0, EB)],
                              hgs[slot], shs[slot]).wait()

        @pl.when(gblk >= 2)
        def _():
            pltpu.make_async_copy(zeros_hbm.at[pl.ds(0, EB)],
                                  ms[slot], sss[slot]).wait()

        ta_v = tas[slot]
        hg_v = hgs[slot]
        m_v = ms[slot]

        @plsc.parallel_loop(0, EB, 1, unroll=4)
        def _mul(rr):
            for ch2 in range(D // 32):
                sl = pl.ds(ch2 * 32, 32)
                a1, a2 = plsc.unpack(ta_v[rr, sl],
                                     format=plsc.PackFormat.INTERLEAVED)
                b1, b2 = plsc.unpack(hg_v[rr, sl],
                                     format=plsc.PackFormat.INTERLEAVED)
                m_v[rr, pl.ds(ch2 * 32, 16)] = a1 * b1
                m_v[rr, pl.ds(ch2 * 32 + 16, 16)] = a2 * b2

        nxt = blk + 2

        @pl.when(nxt < CB)
        def _():
            fire(nxt, slot)

        pltpu.async_copy(m_v, agg_sh.at[di2.at[blk]], sss[slot], add=True)

    def chunk(ch, carry):
        cbase = base0 + ch * CH
        pltpu.sync_copy(ki_hbm.at[pl.ds(cbase, CH)], ki_c)
        pltpu.sync_copy(src_hbm.at[pl.ds(cbase, CH)], si_c)
        pltpu.sync_copy(dst2_hbm.at[pl.ds(wid * NBLK + ch * CB, CB)], di2)
        fire(0, 0)
        fire(1, 1)

        def g_loop(g, c2):
            for slot in range(2):
                blk = 2 * g + slot
                drain_and_process(blk, ch * CB + blk, slot)
            return c2

        lax.fori_loop(0, CB // 2, g_loop, 0)
        if CB % 2 == 1:
            drain_and_process(CB - 1, ch * CB + CB - 1, 0)
        return carry

    lax.fori_loop(0, NCH, chunk, 0)
    pltpu.make_async_copy(zeros_hbm.at[pl.ds(0, EB)], m0, ss0).wait()
    pltpu.make_async_copy(zeros_hbm.at[pl.ds(0, EB)], m1, ss1).wait()
    plsc.subcore_barrier()
    pltpu.sync_copy(agg_sh.at[pl.ds(s * NPT, NPT)],
                    out_hbm.at[c, pl.ds(s * NPT, NPT)])


def _table_body(wr1_ref, wr2_ref, wr3_ref, t0_ref, t1_ref):
    i = pl.program_id(0)
    row0 = i * 512
    ridx = (lax.broadcasted_iota(jnp.int32, (512, 1), 0) + row0)
    r = ridx.astype(jnp.float32) * TAB_H
    rs = jnp.maximum(r, 1e-9)
    n = (lax.broadcasted_iota(jnp.int32, (1, NB), 1) + 1).astype(jnp.float32)
    rb = (np.float32(np.sqrt(2.0 / RMAX))
          * jnp.sin(rs * (np.pi / RMAX) * n) / rs)
    x = r * np.float32(1.0 / RMAX)
    x2 = x * x
    x5 = x2 * x2 * x
    env = (1.0 - 21.0 * x5 + 35.0 * x5 * x - 15.0 * x5 * x2)
    env = jnp.where(x < 1.0, env, 0.0)
    rb = rb * env
    outs = (t0_ref, t1_ref)
    for i2 in range(NI):
        t = rb @ wr1_ref[i2]
        t = t * jax.nn.sigmoid(t)
        t = t @ wr2_ref[i2]
        t = t * jax.nn.sigmoid(t)
        outs[i2][:, :] = (t @ wr3_ref[i2]).astype(jnp.bfloat16)


def _table(Wr1, Wr2, Wr3):
    return pl.pallas_call(
        _table_body,
        grid=(KTAB // 512,),
        in_specs=[
            pl.BlockSpec((NI, NB, H1), lambda i: (0, 0, 0)),
            pl.BlockSpec((NI, H1, H1), lambda i: (0, 0, 0)),
            pl.BlockSpec((NI, H1, D), lambda i: (0, 0, 0)),
        ],
        out_specs=[
            pl.BlockSpec((512, D), lambda i: (i, 0)),
            pl.BlockSpec((512, D), lambda i: (i, 0)),
        ],
        out_shape=[
            jax.ShapeDtypeStruct((KTAB, D), jnp.bfloat16),
            jax.ShapeDtypeStruct((KTAB, D), jnp.bfloat16),
        ],
    )(Wr1, Wr2, Wr3)


def _node_init_body(sp_ref, wemb_ref, e0_ref, wup0_ref, h_ref, hs_ref, en_ref):
    sp = sp_ref[:, :]
    zi = lax.broadcasted_iota(jnp.int32, (1, Z), 1)
    oh = (sp == zi).astype(jnp.float32)
    h = oh @ wemb_ref[:, :]
    h_ref[:, :] = h
    hs_ref[:, :] = (h @ wup0_ref[:, :]).astype(jnp.bfloat16)
    en_ref[:, :] = oh @ e0_ref[:, :]


def _node_init(species2, W_embed, E0c, Wup0, bn=2000):
    grid = N // bn
    return pl.pallas_call(
        _node_init_body,
        grid=(grid,),
        in_specs=[
            pl.BlockSpec((bn, 1), lambda i: (i, 0)),
            pl.BlockSpec((Z, D), lambda i: (0, 0)),
            pl.BlockSpec((Z, 1), lambda i: (0, 0)),
            pl.BlockSpec((D, D), lambda i: (0, 0)),
        ],
        out_specs=[
            pl.BlockSpec((bn, D), lambda i: (i, 0)),
            pl.BlockSpec((bn, D), lambda i: (i, 0)),
            pl.BlockSpec((bn, 1), lambda i: (i, 0)),
        ],
        out_shape=[
            jax.ShapeDtypeStruct((N, D), jnp.float32),
            jax.ShapeDtypeStruct((N, D), jnp.bfloat16),
            jax.ShapeDtypeStruct((N, 1), jnp.float32),
        ],
    )(species2, W_embed, E0c, Wup0)


def _node_update_body(final, agg_ref, h_ref, sp_ref, en_ref, wout_ref,
                      wskip_ref, wprod_ref, wpl_ref, wro_ref, wup_ref,
                      wh_ref, wo_ref, h_out, hs_out, en_out):
    agg = (agg_ref[0] + agg_ref[1]) * np.float32(1.0 / AVG)
    h1 = agg @ wout_ref[:, :] + h_ref[:, :] @ wskip_ref[:, :]
    sp = sp_ref[:, :]
    zi = lax.broadcasted_iota(jnp.int32, (1, Z), 1)
    oh = (sp == zi).astype(jnp.float32)
    w = oh @ wprod_ref[:, :]
    g = (w[:, 0:D] * h1 + w[:, D:2 * D] * (h1 * h1)
         + w[:, 2 * D:3 * D] * (h1 * h1 * h1))
    h2 = g @ wpl_ref[:, :]
    h_out[:, :] = h2
    if final:
        t = h2 @ wh_ref[:, :]
        t = t * jax.nn.sigmoid(t)
        en_out[:, :] = en_ref[:, :] + t @ wo_ref[:, :]
        hs_out[:, :] = h2.astype(jnp.bfloat16)
    else:
        en_out[:, :] = en_ref[:, :] + h2 @ wro_ref[:, :]
        hs_out[:, :] = (h2 @ wup_ref[:, :]).astype(jnp.bfloat16)


def _node_update(final, agg2, h, species2, en, Wout_i, Wskip_i, WprodF_i,
                 WprodLin_i, Wro_i, Wup_n, Wh, Wo, bn=2000):
    grid = N // bn
    return pl.pallas_call(
        functools.partial(_node_update_body, final),
        grid=(grid,),
        in_specs=[
            pl.BlockSpec((NC, bn, D), lambda i: (0, i, 0)),
            pl.BlockSpec((bn, D), lambda i: (i, 0)),
            pl.BlockSpec((bn, 1), lambda i: (i, 0)),
            pl.BlockSpec((bn, 1), lambda i: (i, 0)),
            pl.BlockSpec((D, D), lambda i: (0, 0)),
            pl.BlockSpec((D, D), lambda i: (0, 0)),
            pl.BlockSpec((Z, 3 * D), lambda i: (0, 0)),
            pl.BlockSpec((D, D), lambda i: (0, 0)),
            pl.BlockSpec((D, 1), lambda i: (0, 0)),
            pl.BlockSpec((D, D), lambda i: (0, 0)),
            pl.BlockSpec((D, MLPH), lambda i: (0, 0)),
            pl.BlockSpec((MLPH, 1), lambda i: (0, 0)),
        ],
        out_specs=[
            pl.BlockSpec((bn, D), lambda i: (i, 0)),
            pl.BlockSpec((bn, D), lambda i: (i, 0)),
            pl.BlockSpec((bn, 1), lambda i: (i, 0)),
        ],
        out_shape=[
            jax.ShapeDtypeStruct((N, D), jnp.float32),
            jax.ShapeDtypeStruct((N, D), jnp.bfloat16),
            jax.ShapeDtypeStruct((N, 1), jnp.float32),
        ],
    )(agg2, h, species2, en, Wout_i, Wskip_i, WprodF_i, WprodLin_i,
      Wro_i, Wup_n, Wh, Wo)


def kernel(positions, species, edge_index, W_embed, E0, Wr1, Wr2, Wr3,
           Wup, Wout, Wskip, Wprod, WprodLin, Wro, Wh, Wo):
    src = edge_index[0]
    dst = edge_index[1]
    px = positions[:, 0]
    py = positions[:, 1]
    pz = positions[:, 2]
    species2 = species.reshape(N, 1).astype(jnp.int32)
    zeros_nd = jnp.zeros((N, D), jnp.float32)

    Wr3p = Wr3[:, :, PRM]
    Wupp = Wup[:, :, PRM]
    kidx = _geom(px, py, pz, src, dst)
    t0, t1 = _table(Wr1, Wr2, Wr3p)
    h, hs, en = _node_init(species2, W_embed, E0.reshape(Z, 1), Wupp[0])

    dst2 = dst.reshape(E // EB, EB)
    tabs = (t0, t1)
    for i in range(NI):
        agg2 = _gather_mul_scatter(tabs[i], hs, kidx, src, dst2, zeros_nd)
        h, hs, en = _node_update(
            i == NI - 1, agg2, h, species2, en,
            Wout[i], Wskip[i], Wprod[i].reshape(Z, 3 * D), WprodLin[i],
            Wro[i], Wupp[(i + 1) % NI], Wh, Wo)
    return en.reshape(N)
